# baseline (speedup 1.0000x reference)
import functools
import math

import numpy as np
import jax
import jax.numpy as jnp
from jax import lax
from jax.experimental import pallas as pl
from jax.experimental.pallas import tpu as pltpu
from jax.experimental.pallas import tpu_sc as plsc

_LANES = 16


def _pos_encoding(seq_len: int, d: int) -> np.ndarray:
    pos = np.arange(seq_len, dtype=np.float32)[:, None]
    fill = pos * np.exp(
        -np.arange(0, d, 2, dtype=np.float32) * math.log(10000.0) / d)
    pe = np.zeros((seq_len, d), dtype=np.float32)
    pe[:, 0::2] = np.sin(fill)
    pe[:, 1::2] = np.cos(fill)
    return pe


def kernel(x, table):
    b, seq = x.shape
    _, d = table.shape
    info = plsc.get_sparse_core_info()
    nw = info.num_cores * info.num_subcores
    bpw = b // nw
    half = seq // 2
    nq = d // _LANES

    xw = x.astype(jnp.int32).reshape(nw, bpw, 2, half)
    pe = jnp.asarray(_pos_encoding(seq, d))

    mesh = plsc.VectorSubcoreMesh(core_axis_name="c", subcore_axis_name="s")

    @functools.partial(
        pl.kernel,
        mesh=mesh,
        out_type=jax.ShapeDtypeStruct((b, seq, d), jnp.float32),
        compiler_params=pltpu.CompilerParams(use_tc_tiling_on_sc=False),
        scratch_types=[
            pltpu.VMEM((bpw, 2, half), jnp.int32),
            pltpu.VMEM((seq, d), jnp.float32),
            pltpu.VMEM((6, seq, d), jnp.float32),
        ] + [pltpu.SemaphoreType.DMA] * 18,
    )
    def run(x_hbm, table_hbm, pe_hbm, out_hbm,
            idx_v, pe_v, rows_v, *sems):
        wid = lax.axis_index("s") * info.num_cores + lax.axis_index("c")
        pltpu.sync_copy(x_hbm.at[wid], idx_v)
        pltpu.sync_copy(pe_hbm, pe_v)

        gsems = tuple((sems[2 * i], sems[2 * i + 1]) for i in range(6))
        ssems = sems[12:18]

        def fire_gather(bi, slot):
            for h in range(2):
                pltpu.async_copy(
                    table_hbm.at[idx_v.at[bi, h]],
                    rows_v.at[slot, pl.ds(h * half, half)],
                    gsems[slot][h])

        def wait_gather(slot):
            for h in range(2):
                pltpu.make_async_copy(
                    table_hbm.at[idx_v.at[0, h]],
                    rows_v.at[slot, pl.ds(h * half, half)],
                    gsems[slot][h]).wait()

        def fire_store(bi, slot):
            pltpu.async_copy(
                rows_v.at[slot], out_hbm.at[wid * bpw + bi], ssems[slot])

        def wait_store(slot):
            pltpu.make_async_copy(
                rows_v.at[slot], out_hbm.at[0], ssems[slot]).wait()

        def add_pe(slot):
            def l_body(l, c):
                for q in range(nq):
                    sl_ = pl.ds(q * _LANES, _LANES)
                    rows_v[slot, l, sl_] = rows_v[slot, l, sl_] + pe_v[l, sl_]
                return c
            lax.fori_loop(0, seq, l_body, 0, unroll=10)

        def step(bi, sl, wait_st, fire_g):
            if wait_st:
                wait_store((sl - 3) % 6)
            if fire_g:
                fire_gather(bi + 3, (sl + 3) % 6)
            wait_gather(sl)
            add_pe(sl)
            fire_store(bi, sl)

        for sl in range(3):
            fire_gather(sl, sl)
        for bi in range(6):
            step(bi, bi % 6, wait_st=(bi >= 3), fire_g=True)

        def hex_body(t, carry):
            for sl in range(6):
                step(t * 6 + sl, sl, wait_st=True, fire_g=True)
            return carry

        lax.fori_loop(1, (bpw - 8) // 6, hex_body, 0)

        for bi in range(bpw - 8, bpw):
            step(bi, bi % 6, wait_st=True, fire_g=(bi + 3 < bpw))
        for bi in range(bpw - 3, bpw):
            wait_store(bi % 6)

    return run(xw, table, pe)


# speedup vs baseline: 1.0859x; 1.0859x over previous
import functools
import math

import numpy as np
import jax
import jax.numpy as jnp
from jax import lax
from jax.experimental import pallas as pl
from jax.experimental.pallas import tpu as pltpu
from jax.experimental.pallas import tpu_sc as plsc

_LANES = 16
_CH = 2


def _pos_encoding(seq_len: int, d: int) -> np.ndarray:
    pos = np.arange(seq_len, dtype=np.float32)[:, None]
    fill = pos * np.exp(
        -np.arange(0, d, 2, dtype=np.float32) * math.log(10000.0) / d)
    pe = np.zeros((seq_len, d), dtype=np.float32)
    pe[:, 0::2] = np.sin(fill)
    pe[:, 1::2] = np.cos(fill)
    return pe


def kernel(x, table):
    b, seq = x.shape
    _, d = table.shape
    info = plsc.get_sparse_core_info()
    nw = info.num_cores * info.num_subcores
    bpw = b // nw
    half = seq // 2
    nq = d // _LANES
    nch = bpw // _CH
    blk = 2 * _CH

    xw = x.astype(jnp.int32).reshape(nw, nch, blk * half)
    pe = jnp.asarray(_pos_encoding(seq, d)).reshape(2, half, d)

    mesh = plsc.VectorSubcoreMesh(core_axis_name="c", subcore_axis_name="s")

    @functools.partial(
        pl.kernel,
        mesh=mesh,
        out_type=jax.ShapeDtypeStruct((b * seq, d), jnp.float32),
        compiler_params=pltpu.CompilerParams(use_tc_tiling_on_sc=False),
        scratch_types=[
            pltpu.VMEM((nch, blk * half), jnp.int32),
            pltpu.VMEM((2, half, d), jnp.float32),
            pltpu.VMEM((3, blk * half, d), jnp.float32),
        ] + [pltpu.SemaphoreType.DMA] * 6,
    )
    def run(x_hbm, table_hbm, pe_hbm, out_hbm, idx_v, pe_v, rows_v, *sems):
        wid = lax.axis_index("s") * info.num_cores + lax.axis_index("c")
        pltpu.sync_copy(x_hbm.at[wid], idx_v)
        pltpu.sync_copy(pe_hbm, pe_v)

        gsems = sems[0:3]
        ssems = sems[3:6]

        def fire_gather(ci, slot):
            pltpu.async_copy(
                table_hbm.at[idx_v.at[ci]], rows_v.at[slot], gsems[slot])

        def wait_gather(slot):
            pltpu.make_async_copy(
                table_hbm.at[idx_v.at[0]], rows_v.at[slot],
                gsems[slot]).wait()

        def fire_store(ci, slot):
            pltpu.async_copy(
                rows_v.at[slot],
                out_hbm.at[pl.ds(wid * bpw * seq + ci * blk * half, blk * half)],
                ssems[slot])

        def wait_store(slot):
            pltpu.make_async_copy(
                rows_v.at[slot], out_hbm.at[pl.ds(0, blk * half)],
                ssems[slot]).wait()

        def add_pe(slot):
            def r_body(r, c):
                for j in range(blk):
                    for q in range(nq):
                        sl_ = pl.ds(q * _LANES, _LANES)
                        rows_v[slot, j * half + r, sl_] = (
                            rows_v[slot, j * half + r, sl_] + pe_v[j % 2, r, sl_])
                return c
            lax.fori_loop(0, half, r_body, 0, unroll=4)

        def step(ci, sl, wait_st, fire_g):
            if wait_st:
                wait_store((sl + 1) % 3)
            if fire_g:
                fire_gather(ci + 1, (sl + 1) % 3)
            wait_gather(sl)
            add_pe(sl)
            fire_store(ci, sl)

        fire_gather(0, 0)
        for ci in range(3):
            step(ci, ci % 3, wait_st=(ci >= 2), fire_g=True)

        def tri_body(t, carry):
            for sl in range(3):
                step(t * 3 + sl, sl, wait_st=True, fire_g=True)
            return carry

        lax.fori_loop(1, (nch - 1) // 3, tri_body, 0)

        for ci in range(((nch - 1) // 3) * 3, nch):
            step(ci, ci % 3, wait_st=(ci + 1 < nch), fire_g=(ci + 1 < nch))
        for ci in range(nch - 3, nch):
            wait_store(ci % 3)

    out = run(xw, table, pe)
    return out.reshape(b, seq, d)
